# baseline (speedup 1.0000x reference)
import jax
import jax.numpy as jnp
from jax import lax
from jax.experimental import pallas as pl
from jax.experimental.pallas import tpu as pltpu
from jax.experimental.pallas import tpu_sc as plsc

N_NODES = 10000
N_EDGES = 320000
D = 128
_f32 = jnp.float32

_NC, _NS = 2, 16
_NW = _NC * _NS
_EPW = N_EDGES // _NW

_GCH = 80
_GCHUNK = _GCH
_GNCH = _EPW // _GCHUNK
_GIPT = _GNCH

_SCH = 80
_SIPT = _EPW // _SCH


def _sc_mesh():
    return plsc.VectorSubcoreMesh(
        core_axis_name="c", subcore_axis_name="s", num_cores=_NC, num_subcores=_NS
    )


def _gather_add_body(p_hbm, q_hbm, src3_hbm, dst3_hbm, g_hbm,
                     idx_s, idx_d, pA, pB, qA, qB, semA, semB, semwA, semwB):
    c = lax.axis_index("c")
    s = lax.axis_index("s")
    wid = s * _NC + c
    base = wid * _EPW

    pltpu.sync_copy(src3_hbm.at[wid], idx_s)
    pltpu.sync_copy(dst3_hbm.at[wid], idx_d)

    def g_start(j, bufp, bufq, sem):
        pltpu.async_copy(p_hbm.at[idx_s.at[j]], bufp, sem)
        pltpu.async_copy(q_hbm.at[idx_d.at[j]], bufq, sem)

    def g_wait(bufp, bufq, sem):
        pltpu.make_async_copy(g_hbm.at[pl.ds(0, _GCHUNK)], bufp, sem).wait()
        pltpu.make_async_copy(g_hbm.at[pl.ds(0, _GCHUNK)], bufq, sem).wait()

    def add(bufp, bufq):
        def rows(r, carry):
            for u in range(2):
                for v in range(D // 16):
                    sl = pl.ds(v * 16, 16)
                    plsc.addupdate(bufp.at[2 * r + u, sl], bufq[2 * r + u, sl])
            return carry

        lax.fori_loop(0, _GCHUNK // 2, rows, 0)

    def wb_start(gc, bufp, semw):
        pltpu.async_copy(bufp, g_hbm.at[pl.ds(base + gc * _GCHUNK, _GCHUNK)],
                         semw)

    def wb_wait(bufp, semw):
        pltpu.make_async_copy(bufp, g_hbm.at[pl.ds(0, _GCHUNK)], semw).wait()

    g_start(0, pA, qA, semA)
    g_start(1, pB, qB, semB)

    def body(t, carry):
        c0 = 2 * t
        g_wait(pA, qA, semA)
        add(pA, qA)
        wb_start(c0, pA, semwA)
        g_wait(pB, qB, semB)
        add(pB, qB)
        wb_start(c0 + 1, pB, semwB)
        wb_wait(pA, semwA)
        g_start(c0 + 2, pA, qA, semA)

        @pl.when(c0 + 3 < _GNCH)
        def _():
            wb_wait(pB, semwB)
            g_start(c0 + 3, pB, qB, semB)

        return carry

    lax.fori_loop(0, _GNCH // 2, body, 0)
    g_wait(pA, qA, semA)
    add(pA, qA)
    wb_start(_GNCH - 1, pA, semwA)
    wb_wait(pB, semwB)
    wb_wait(pA, semwA)


def _gather_add(p, q, src3, dst3):
    f = pl.kernel(
        _gather_add_body,
        out_type=jax.ShapeDtypeStruct((N_EDGES, D), _f32),
        mesh=_sc_mesh(),
        scratch_types=(
            pltpu.VMEM((_GIPT, _GCH), jnp.int32),
            pltpu.VMEM((_GIPT, _GCH), jnp.int32),
            pltpu.VMEM((_GCHUNK, D), _f32),
            pltpu.VMEM((_GCHUNK, D), _f32),
            pltpu.VMEM((_GCHUNK, D), _f32),
            pltpu.VMEM((_GCHUNK, D), _f32),
            pltpu.SemaphoreType.DMA,
            pltpu.SemaphoreType.DMA,
            pltpu.SemaphoreType.DMA,
            pltpu.SemaphoreType.DMA,
        ),
    )
    return f(p, q, src3, dst3)


def _scatter_body(edges_hbm, dst3_hbm, zeros_hbm, out_hbm,
                  idx_d, bufA, bufB, acc, semA, semB):
    c = lax.axis_index("c")
    s = lax.axis_index("s")
    wid = c * _NS + s
    base = wid * _EPW

    pltpu.sync_copy(dst3_hbm.at[wid], idx_d)

    @pl.when(s == 0)
    def _():
        pltpu.sync_copy(zeros_hbm, acc)

    plsc.subcore_barrier()

    def r_start(ch, buf, sem):
        pltpu.async_copy(edges_hbm.at[pl.ds(base + ch * _SCH, _SCH)], buf, sem)

    def r_wait(buf, sem):
        pltpu.make_async_copy(edges_hbm.at[pl.ds(0, _SCH)], buf, sem).wait()

    def scat(ch, buf):
        pltpu.sync_copy(buf, acc.at[idx_d.at[ch]], add=True)

    r_start(0, bufA, semA)
    r_start(1, bufB, semB)

    def body(t, carry):
        c0 = 2 * t
        r_wait(bufA, semA)
        scat(c0, bufA)
        r_start(c0 + 2, bufA, semA)
        r_wait(bufB, semB)
        scat(c0 + 1, bufB)

        @pl.when(c0 + 3 < _SIPT)
        def _():
            r_start(c0 + 3, bufB, semB)

        return carry

    lax.fori_loop(0, _SIPT // 2, body, 0)
    r_wait(bufA, semA)
    scat(_SIPT - 1, bufA)

    plsc.subcore_barrier()

    @pl.when(s == 0)
    def _():
        pltpu.sync_copy(acc, out_hbm.at[c])


def _scatter_add(edges, dst3, zeros):
    f = pl.kernel(
        _scatter_body,
        out_type=jax.ShapeDtypeStruct((_NC, N_NODES, D), _f32),
        mesh=_sc_mesh(),
        scratch_types=(
            pltpu.VMEM((_SIPT, _SCH), jnp.int32),
            pltpu.VMEM((_SCH, D), _f32),
            pltpu.VMEM((_SCH, D), _f32),
            pltpu.VMEM_SHARED((N_NODES, D), _f32),
            pltpu.SemaphoreType.DMA,
            pltpu.SemaphoreType.DMA,
        ),
    )
    return f(edges, dst3, zeros)


_NBLK = 1000
_NGRID = N_NODES // _NBLK
_EBLK = 2560
_EGRID = N_EDGES // _EBLK


def _row_spec(blk):
    return pl.BlockSpec((blk, D), lambda i: (i, 0))


def _w_spec():
    return pl.BlockSpec((D, D), lambda i: (0, 0))


def _b_spec():
    return pl.BlockSpec((1, D), lambda i: (0, 0))


def _dot(a, b):
    return jnp.dot(a, b, preferred_element_type=_f32)


def _enc_body(x_ref, w1_ref, b1_ref, w2_ref, b2_ref, a_ref, b_ref,
              nodes_ref, p_ref, q_ref):
    h = jax.nn.gelu(_dot(x_ref[...], w1_ref[...]) + b1_ref[...])
    n = _dot(h, w2_ref[...]) + b2_ref[...]
    nodes_ref[...] = n
    p_ref[...] = _dot(n, a_ref[...])
    q_ref[...] = _dot(n, b_ref[...])


def _node_enc(x, w1, b1, w2, b2, a, b):
    return pl.pallas_call(
        _enc_body,
        grid=(_NGRID,),
        in_specs=[_row_spec(_NBLK), _w_spec(), _b_spec(), _w_spec(), _b_spec(),
                  _w_spec(), _w_spec()],
        out_specs=[_row_spec(_NBLK)] * 3,
        out_shape=[jax.ShapeDtypeStruct((N_NODES, D), _f32)] * 3,
    )(x, w1, b1, w2, b2, a, b)


def _node_layer_body(n_ref, agg_ref, wn_ref, wa_ref, b1_ref, w2_ref, b2_ref,
                     a_ref, b_ref, out_ref, p_ref, q_ref):
    n = n_ref[...]
    agg = agg_ref[0] + agg_ref[1]
    h = jax.nn.gelu(_dot(n, wn_ref[...]) + _dot(agg, wa_ref[...]) + b1_ref[...])
    nn = _dot(h, w2_ref[...]) + b2_ref[...] + n
    out_ref[...] = nn
    p_ref[...] = _dot(nn, a_ref[...])
    q_ref[...] = _dot(nn, b_ref[...])


def _node_layer(nodes, agg2, wn, wa, b1, w2, b2, a, b):
    return pl.pallas_call(
        _node_layer_body,
        grid=(_NGRID,),
        in_specs=[_row_spec(_NBLK),
                  pl.BlockSpec((_NC, _NBLK, D), lambda i: (0, i, 0)),
                  _w_spec(), _w_spec(), _b_spec(), _w_spec(), _b_spec(),
                  _w_spec(), _w_spec()],
        out_specs=[_row_spec(_NBLK)] * 3,
        out_shape=[jax.ShapeDtypeStruct((N_NODES, D), _f32)] * 3,
    )(nodes, agg2, wn, wa, b1, w2, b2, a, b)


def _edge_enc_body(g_ref, b1_ref, w2_ref, b2_ref, out_ref):
    h = jax.nn.gelu(g_ref[...] + b1_ref[...])
    out_ref[...] = _dot(h, w2_ref[...]) + b2_ref[...]


def _edge_enc(g, b1, w2, b2):
    return pl.pallas_call(
        _edge_enc_body,
        grid=(_EGRID,),
        in_specs=[_row_spec(_EBLK), _b_spec(), _w_spec(), _b_spec()],
        out_specs=_row_spec(_EBLK),
        out_shape=jax.ShapeDtypeStruct((N_EDGES, D), _f32),
    )(g, b1, w2, b2)


def _edge_layer_body(e_ref, g_ref, c_ref, b1_ref, w2_ref, b2_ref, out_ref):
    e = e_ref[...]
    h = jax.nn.gelu(g_ref[...] + _dot(e, c_ref[...]) + b1_ref[...])
    out_ref[...] = _dot(h, w2_ref[...]) + b2_ref[...] + e


def _edge_layer(edges, g, c, b1, w2, b2):
    return pl.pallas_call(
        _edge_layer_body,
        grid=(_EGRID,),
        in_specs=[_row_spec(_EBLK), _row_spec(_EBLK), _w_spec(), _b_spec(),
                  _w_spec(), _b_spec()],
        out_specs=_row_spec(_EBLK),
        out_shape=jax.ShapeDtypeStruct((N_EDGES, D), _f32),
    )(edges, g, c, b1, w2, b2)


def kernel(node_attr, graph, params):
    src = graph[0].astype(jnp.int32)
    dst = graph[1].astype(jnp.int32)
    src3 = src.reshape(_NW, _GIPT, _GCH)
    dst3 = dst.reshape(_NW, _GIPT, _GCH)
    enc = params["node_enc"]
    ee = params["edge_enc"]

    nodes, p, q = _node_enc(
        node_attr, enc["W1"], enc["b1"].reshape(1, D), enc["W2"],
        enc["b2"].reshape(1, D), ee["W1"][:D], ee["W1"][D:])
    g = _gather_add(p, q, src3, dst3)
    edges = _edge_enc(g, ee["b1"].reshape(1, D), ee["W2"],
                      ee["b2"].reshape(1, D))

    zeros = jnp.zeros((N_NODES, D), _f32)
    for lp in params["layers"]:
        np_ = lp["node_net"]
        ep_ = lp["edge_net"]
        agg2 = _scatter_add(edges, dst3, zeros)
        nodes, p, q = _node_layer(
            nodes, agg2, np_["W1"][:D], np_["W1"][D:],
            np_["b1"].reshape(1, D), np_["W2"], np_["b2"].reshape(1, D),
            ep_["W1"][:D], ep_["W1"][D:2 * D])
        g = _gather_add(p, q, src3, dst3)
        edges = _edge_layer(edges, g, ep_["W1"][2 * D:],
                            ep_["b1"].reshape(1, D), ep_["W2"],
                            ep_["b2"].reshape(1, D))
    return (nodes, edges)


# speedup vs baseline: 1.0591x; 1.0591x over previous
import jax
import jax.numpy as jnp
from jax import lax
from jax.experimental import pallas as pl
from jax.experimental.pallas import tpu as pltpu
from jax.experimental.pallas import tpu_sc as plsc

N_NODES = 10000
N_EDGES = 320000
D = 128
_f32 = jnp.float32

_NC, _NS = 2, 16
_NW = _NC * _NS
_EPW = N_EDGES // _NW

_GCH = 80
_GCHUNK = _GCH
_GNCH = _EPW // _GCHUNK
_GIPT = _GNCH

_SCH = 80
_SIPT = _EPW // _SCH


def _sc_mesh():
    return plsc.VectorSubcoreMesh(
        core_axis_name="c", subcore_axis_name="s", num_cores=_NC, num_subcores=_NS
    )


def _gather_add_body(p_hbm, q_hbm, src3_hbm, dst3_hbm, g_hbm,
                     idx_s, idx_d, pA, pB, qA, qB, semA, semB, semwA, semwB):
    c = lax.axis_index("c")
    s = lax.axis_index("s")
    wid = s * _NC + c
    base = wid * _EPW

    pltpu.sync_copy(src3_hbm.at[wid], idx_s)
    pltpu.sync_copy(dst3_hbm.at[wid], idx_d)

    def g_start(j, bufp, bufq, sem):
        pltpu.async_copy(p_hbm.at[idx_s.at[j]], bufp, sem)
        pltpu.async_copy(q_hbm.at[idx_d.at[j]], bufq, sem)

    def g_wait(bufp, bufq, sem):
        pltpu.make_async_copy(g_hbm.at[pl.ds(0, _GCHUNK)], bufp, sem).wait()
        pltpu.make_async_copy(g_hbm.at[pl.ds(0, _GCHUNK)], bufq, sem).wait()

    def add(bufp, bufq):
        def rows(r, carry):
            for u in range(2):
                for v in range(D // 16):
                    sl = pl.ds(v * 16, 16)
                    plsc.addupdate(bufp.at[2 * r + u, sl], bufq[2 * r + u, sl])
            return carry

        lax.fori_loop(0, _GCHUNK // 2, rows, 0)

    def wb_start(gc, bufp, semw):
        pltpu.async_copy(bufp, g_hbm.at[pl.ds(base + gc * _GCHUNK, _GCHUNK)],
                         semw)

    def wb_wait(bufp, semw):
        pltpu.make_async_copy(bufp, g_hbm.at[pl.ds(0, _GCHUNK)], semw).wait()

    g_start(0, pA, qA, semA)
    g_start(1, pB, qB, semB)

    def body(t, carry):
        c0 = 2 * t
        g_wait(pA, qA, semA)
        add(pA, qA)
        wb_start(c0, pA, semwA)
        g_wait(pB, qB, semB)
        add(pB, qB)
        wb_start(c0 + 1, pB, semwB)
        wb_wait(pA, semwA)
        g_start(c0 + 2, pA, qA, semA)

        @pl.when(c0 + 3 < _GNCH)
        def _():
            wb_wait(pB, semwB)
            g_start(c0 + 3, pB, qB, semB)

        return carry

    lax.fori_loop(0, _GNCH // 2, body, 0)
    g_wait(pA, qA, semA)
    add(pA, qA)
    wb_start(_GNCH - 1, pA, semwA)
    wb_wait(pB, semwB)
    wb_wait(pA, semwA)


def _gather_add(p, q, src3, dst3):
    f = pl.kernel(
        _gather_add_body,
        out_type=jax.ShapeDtypeStruct((N_EDGES, D), _f32),
        mesh=_sc_mesh(),
        scratch_types=(
            pltpu.VMEM((_GIPT, _GCH), jnp.int32),
            pltpu.VMEM((_GIPT, _GCH), jnp.int32),
            pltpu.VMEM((_GCHUNK, D), _f32),
            pltpu.VMEM((_GCHUNK, D), _f32),
            pltpu.VMEM((_GCHUNK, D), _f32),
            pltpu.VMEM((_GCHUNK, D), _f32),
            pltpu.SemaphoreType.DMA,
            pltpu.SemaphoreType.DMA,
            pltpu.SemaphoreType.DMA,
            pltpu.SemaphoreType.DMA,
        ),
    )
    return f(p, q, src3, dst3)


def _scatter_body(edges_hbm, dst3_hbm, zeros_hbm, out_hbm,
                  idx_d, bufA, bufB, acc, semA, semB):
    c = lax.axis_index("c")
    s = lax.axis_index("s")
    wid = c * _NS + s
    base = wid * _EPW

    pltpu.sync_copy(dst3_hbm.at[wid], idx_d)

    @pl.when(s == 0)
    def _():
        pltpu.sync_copy(zeros_hbm, acc)

    plsc.subcore_barrier()

    def r_start(ch, buf, sem):
        pltpu.async_copy(edges_hbm.at[pl.ds(base + ch * _SCH, _SCH)], buf, sem)

    def r_wait(buf, sem):
        pltpu.make_async_copy(edges_hbm.at[pl.ds(0, _SCH)], buf, sem).wait()

    def scat(ch, buf):
        pltpu.sync_copy(buf, acc.at[idx_d.at[ch]], add=True)

    r_start(0, bufA, semA)
    r_start(1, bufB, semB)

    def body(t, carry):
        c0 = 2 * t
        r_wait(bufA, semA)
        scat(c0, bufA)
        r_start(c0 + 2, bufA, semA)
        r_wait(bufB, semB)
        scat(c0 + 1, bufB)

        @pl.when(c0 + 3 < _SIPT)
        def _():
            r_start(c0 + 3, bufB, semB)

        return carry

    lax.fori_loop(0, _SIPT // 2, body, 0)
    r_wait(bufA, semA)
    scat(_SIPT - 1, bufA)

    plsc.subcore_barrier()

    @pl.when(s == 0)
    def _():
        pltpu.sync_copy(acc, out_hbm.at[c])


def _scatter_add(edges, dst3, zeros):
    f = pl.kernel(
        _scatter_body,
        out_type=jax.ShapeDtypeStruct((_NC, N_NODES, D), _f32),
        mesh=_sc_mesh(),
        scratch_types=(
            pltpu.VMEM((_SIPT, _SCH), jnp.int32),
            pltpu.VMEM((_SCH, D), _f32),
            pltpu.VMEM((_SCH, D), _f32),
            pltpu.VMEM_SHARED((N_NODES, D), _f32),
            pltpu.SemaphoreType.DMA,
            pltpu.SemaphoreType.DMA,
        ),
    )
    return f(edges, dst3, zeros)


_NBLK = 1000
_NGRID = N_NODES // _NBLK
_EBLK = 4000
_EGRID = N_EDGES // _EBLK


def _row_spec(blk):
    return pl.BlockSpec((blk, D), lambda i: (i, 0))


def _w_spec():
    return pl.BlockSpec((D, D), lambda i: (0, 0))


def _b_spec():
    return pl.BlockSpec((1, D), lambda i: (0, 0))


def _dot(a, b):
    return jnp.dot(a, b, preferred_element_type=_f32)


def _enc_body(x_ref, w1_ref, b1_ref, w2_ref, b2_ref, a_ref, b_ref,
              nodes_ref, p_ref, q_ref):
    h = jax.nn.gelu(_dot(x_ref[...], w1_ref[...]) + b1_ref[...])
    n = _dot(h, w2_ref[...]) + b2_ref[...]
    nodes_ref[...] = n
    p_ref[...] = _dot(n, a_ref[...])
    q_ref[...] = _dot(n, b_ref[...])


def _node_enc(x, w1, b1, w2, b2, a, b):
    return pl.pallas_call(
        _enc_body,
        grid=(_NGRID,),
        in_specs=[_row_spec(_NBLK), _w_spec(), _b_spec(), _w_spec(), _b_spec(),
                  _w_spec(), _w_spec()],
        out_specs=[_row_spec(_NBLK)] * 3,
        out_shape=[jax.ShapeDtypeStruct((N_NODES, D), _f32)] * 3,
    )(x, w1, b1, w2, b2, a, b)


def _node_layer_body(n_ref, agg_ref, wn_ref, wa_ref, b1_ref, w2_ref, b2_ref,
                     a_ref, b_ref, out_ref, p_ref, q_ref):
    n = n_ref[...]
    agg = agg_ref[0] + agg_ref[1]
    h = jax.nn.gelu(_dot(n, wn_ref[...]) + _dot(agg, wa_ref[...]) + b1_ref[...])
    nn = _dot(h, w2_ref[...]) + b2_ref[...] + n
    out_ref[...] = nn
    p_ref[...] = _dot(nn, a_ref[...])
    q_ref[...] = _dot(nn, b_ref[...])


def _node_layer(nodes, agg2, wn, wa, b1, w2, b2, a, b):
    return pl.pallas_call(
        _node_layer_body,
        grid=(_NGRID,),
        in_specs=[_row_spec(_NBLK),
                  pl.BlockSpec((_NC, _NBLK, D), lambda i: (0, i, 0)),
                  _w_spec(), _w_spec(), _b_spec(), _w_spec(), _b_spec(),
                  _w_spec(), _w_spec()],
        out_specs=[_row_spec(_NBLK)] * 3,
        out_shape=[jax.ShapeDtypeStruct((N_NODES, D), _f32)] * 3,
    )(nodes, agg2, wn, wa, b1, w2, b2, a, b)


def _edge_enc_body(g_ref, b1_ref, w2_ref, b2_ref, out_ref):
    h = jax.nn.gelu(g_ref[...] + b1_ref[...])
    out_ref[...] = _dot(h, w2_ref[...]) + b2_ref[...]


def _edge_enc(g, b1, w2, b2):
    return pl.pallas_call(
        _edge_enc_body,
        grid=(_EGRID,),
        in_specs=[_row_spec(_EBLK), _b_spec(), _w_spec(), _b_spec()],
        out_specs=_row_spec(_EBLK),
        out_shape=jax.ShapeDtypeStruct((N_EDGES, D), _f32),
    )(g, b1, w2, b2)


def _edge_layer_body(e_ref, g_ref, c_ref, b1_ref, w2_ref, b2_ref, out_ref):
    e = e_ref[...]
    h = jax.nn.gelu(g_ref[...] + _dot(e, c_ref[...]) + b1_ref[...])
    out_ref[...] = _dot(h, w2_ref[...]) + b2_ref[...] + e


def _edge_layer(edges, g, c, b1, w2, b2):
    return pl.pallas_call(
        _edge_layer_body,
        grid=(_EGRID,),
        in_specs=[_row_spec(_EBLK), _row_spec(_EBLK), _w_spec(), _b_spec(),
                  _w_spec(), _b_spec()],
        out_specs=_row_spec(_EBLK),
        out_shape=jax.ShapeDtypeStruct((N_EDGES, D), _f32),
    )(edges, g, c, b1, w2, b2)


def kernel(node_attr, graph, params):
    src = graph[0].astype(jnp.int32)
    dst = graph[1].astype(jnp.int32)
    src3 = src.reshape(_NW, _GIPT, _GCH)
    dst3 = dst.reshape(_NW, _GIPT, _GCH)
    enc = params["node_enc"]
    ee = params["edge_enc"]

    nodes, p, q = _node_enc(
        node_attr, enc["W1"], enc["b1"].reshape(1, D), enc["W2"],
        enc["b2"].reshape(1, D), ee["W1"][:D], ee["W1"][D:])
    g = _gather_add(p, q, src3, dst3)
    edges = _edge_enc(g, ee["b1"].reshape(1, D), ee["W2"],
                      ee["b2"].reshape(1, D))

    zeros = jnp.zeros((N_NODES, D), _f32)
    for lp in params["layers"]:
        np_ = lp["node_net"]
        ep_ = lp["edge_net"]
        agg2 = _scatter_add(edges, dst3, zeros)
        nodes, p, q = _node_layer(
            nodes, agg2, np_["W1"][:D], np_["W1"][D:],
            np_["b1"].reshape(1, D), np_["W2"], np_["b2"].reshape(1, D),
            ep_["W1"][:D], ep_["W1"][D:2 * D])
        g = _gather_add(p, q, src3, dst3)
        edges = _edge_layer(edges, g, ep_["W1"][2 * D:],
                            ep_["b1"].reshape(1, D), ep_["W2"],
                            ep_["b2"].reshape(1, D))
    return (nodes, edges)


# speedup vs baseline: 1.0917x; 1.0307x over previous
import jax
import jax.numpy as jnp
from jax import lax
from jax.experimental import pallas as pl
from jax.experimental.pallas import tpu as pltpu
from jax.experimental.pallas import tpu_sc as plsc

N_NODES = 10000
N_EDGES = 320000
D = 128
_f32 = jnp.float32

_NC, _NS = 2, 16
_NW = _NC * _NS
_EPW = N_EDGES // _NW

_GCH = 80
_GCHUNK = _GCH
_GNCH = _EPW // _GCHUNK
_GIPT = _GNCH

_SCH = 80
_SIPT = _EPW // _SCH


def _sc_mesh():
    return plsc.VectorSubcoreMesh(
        core_axis_name="c", subcore_axis_name="s", num_cores=_NC, num_subcores=_NS
    )


def _gather_add_body(p_hbm, q_hbm, src3_hbm, dst3_hbm, g_hbm,
                     idx_s, idx_d, pA, pB, qA, qB, semA, semB, semwA, semwB):
    c = lax.axis_index("c")
    s = lax.axis_index("s")
    wid = s * _NC + c
    base = wid * _EPW

    pltpu.sync_copy(src3_hbm.at[wid], idx_s)
    pltpu.sync_copy(dst3_hbm.at[wid], idx_d)

    def g_start(j, bufp, bufq, sem):
        pltpu.async_copy(p_hbm.at[idx_s.at[j]], bufp, sem)
        pltpu.async_copy(q_hbm.at[idx_d.at[j]], bufq, sem)

    def g_wait(bufp, bufq, sem):
        pltpu.make_async_copy(g_hbm.at[pl.ds(0, _GCHUNK)], bufp, sem).wait()
        pltpu.make_async_copy(g_hbm.at[pl.ds(0, _GCHUNK)], bufq, sem).wait()

    def add(bufp, bufq):
        def rows(r, carry):
            for u in range(2):
                for v in range(D // 16):
                    sl = pl.ds(v * 16, 16)
                    plsc.addupdate(bufp.at[2 * r + u, sl], bufq[2 * r + u, sl])
            return carry

        lax.fori_loop(0, _GCHUNK // 2, rows, 0)

    def wb_start(gc, bufp, semw):
        pltpu.async_copy(bufp, g_hbm.at[pl.ds(base + gc * _GCHUNK, _GCHUNK)],
                         semw)

    def wb_wait(bufp, semw):
        pltpu.make_async_copy(bufp, g_hbm.at[pl.ds(0, _GCHUNK)], semw).wait()

    g_start(0, pA, qA, semA)
    g_start(1, pB, qB, semB)

    def body(t, carry):
        c0 = 2 * t
        g_wait(pA, qA, semA)
        add(pA, qA)
        wb_start(c0, pA, semwA)
        g_wait(pB, qB, semB)
        add(pB, qB)
        wb_start(c0 + 1, pB, semwB)
        wb_wait(pA, semwA)
        g_start(c0 + 2, pA, qA, semA)

        @pl.when(c0 + 3 < _GNCH)
        def _():
            wb_wait(pB, semwB)
            g_start(c0 + 3, pB, qB, semB)

        return carry

    lax.fori_loop(0, _GNCH // 2, body, 0)
    g_wait(pA, qA, semA)
    add(pA, qA)
    wb_start(_GNCH - 1, pA, semwA)
    wb_wait(pB, semwB)
    wb_wait(pA, semwA)


def _gather_add(p, q, src3, dst3):
    f = pl.kernel(
        _gather_add_body,
        out_type=jax.ShapeDtypeStruct((N_EDGES, D), _f32),
        mesh=_sc_mesh(),
        scratch_types=(
            pltpu.VMEM((_GIPT, _GCH), jnp.int32),
            pltpu.VMEM((_GIPT, _GCH), jnp.int32),
            pltpu.VMEM((_GCHUNK, D), _f32),
            pltpu.VMEM((_GCHUNK, D), _f32),
            pltpu.VMEM((_GCHUNK, D), _f32),
            pltpu.VMEM((_GCHUNK, D), _f32),
            pltpu.SemaphoreType.DMA,
            pltpu.SemaphoreType.DMA,
            pltpu.SemaphoreType.DMA,
            pltpu.SemaphoreType.DMA,
        ),
    )
    return f(p, q, src3, dst3)


def _scatter_body(edges_hbm, dst3_hbm, zeros_hbm, out_hbm,
                  idx_d, bufA, bufB, acc, semA, semB):
    c = lax.axis_index("c")
    s = lax.axis_index("s")
    wid = c * _NS + s
    base = wid * _EPW

    pltpu.sync_copy(dst3_hbm.at[wid], idx_d)

    @pl.when(s == 0)
    def _():
        pltpu.sync_copy(zeros_hbm, acc)

    plsc.subcore_barrier()

    def r_start(ch, buf, sem):
        pltpu.async_copy(edges_hbm.at[pl.ds(base + ch * _SCH, _SCH)], buf, sem)

    def r_wait(buf, sem):
        pltpu.make_async_copy(edges_hbm.at[pl.ds(0, _SCH)], buf, sem).wait()

    def scat(ch, buf):
        pltpu.sync_copy(buf, acc.at[idx_d.at[ch]], add=True)

    r_start(0, bufA, semA)
    r_start(1, bufB, semB)

    def body(t, carry):
        c0 = 2 * t
        r_wait(bufA, semA)
        scat(c0, bufA)
        r_start(c0 + 2, bufA, semA)
        r_wait(bufB, semB)
        scat(c0 + 1, bufB)

        @pl.when(c0 + 3 < _SIPT)
        def _():
            r_start(c0 + 3, bufB, semB)

        return carry

    lax.fori_loop(0, _SIPT // 2, body, 0)
    r_wait(bufA, semA)
    scat(_SIPT - 1, bufA)

    plsc.subcore_barrier()

    @pl.when(s == 0)
    def _():
        pltpu.sync_copy(acc, out_hbm.at[c])


def _scatter_add(edges, dst3, zeros):
    f = pl.kernel(
        _scatter_body,
        out_type=jax.ShapeDtypeStruct((_NC, N_NODES, D), _f32),
        mesh=_sc_mesh(),
        scratch_types=(
            pltpu.VMEM((_SIPT, _SCH), jnp.int32),
            pltpu.VMEM((_SCH, D), _f32),
            pltpu.VMEM((_SCH, D), _f32),
            pltpu.VMEM_SHARED((N_NODES, D), _f32),
            pltpu.SemaphoreType.DMA,
            pltpu.SemaphoreType.DMA,
        ),
    )
    return f(edges, dst3, zeros)


_NBLK = 1000
_NGRID = N_NODES // _NBLK
_EBLK = 8000
_EGRID = N_EDGES // _EBLK


def _row_spec(blk):
    return pl.BlockSpec((blk, D), lambda i: (i, 0))


def _w_spec():
    return pl.BlockSpec((D, D), lambda i: (0, 0))


def _b_spec():
    return pl.BlockSpec((1, D), lambda i: (0, 0))


def _dot(a, b):
    return jnp.dot(a, b, preferred_element_type=_f32)


def _enc_body(x_ref, w1_ref, b1_ref, w2_ref, b2_ref, a_ref, b_ref,
              nodes_ref, p_ref, q_ref):
    h = jax.nn.gelu(_dot(x_ref[...], w1_ref[...]) + b1_ref[...])
    n = _dot(h, w2_ref[...]) + b2_ref[...]
    nodes_ref[...] = n
    p_ref[...] = _dot(n, a_ref[...])
    q_ref[...] = _dot(n, b_ref[...])


def _node_enc(x, w1, b1, w2, b2, a, b):
    return pl.pallas_call(
        _enc_body,
        grid=(_NGRID,),
        in_specs=[_row_spec(_NBLK), _w_spec(), _b_spec(), _w_spec(), _b_spec(),
                  _w_spec(), _w_spec()],
        out_specs=[_row_spec(_NBLK)] * 3,
        out_shape=[jax.ShapeDtypeStruct((N_NODES, D), _f32)] * 3,
    )(x, w1, b1, w2, b2, a, b)


def _node_layer_body(n_ref, agg_ref, wn_ref, wa_ref, b1_ref, w2_ref, b2_ref,
                     a_ref, b_ref, out_ref, p_ref, q_ref):
    n = n_ref[...]
    agg = agg_ref[0] + agg_ref[1]
    h = jax.nn.gelu(_dot(n, wn_ref[...]) + _dot(agg, wa_ref[...]) + b1_ref[...])
    nn = _dot(h, w2_ref[...]) + b2_ref[...] + n
    out_ref[...] = nn
    p_ref[...] = _dot(nn, a_ref[...])
    q_ref[...] = _dot(nn, b_ref[...])


def _node_layer(nodes, agg2, wn, wa, b1, w2, b2, a, b):
    return pl.pallas_call(
        _node_layer_body,
        grid=(_NGRID,),
        in_specs=[_row_spec(_NBLK),
                  pl.BlockSpec((_NC, _NBLK, D), lambda i: (0, i, 0)),
                  _w_spec(), _w_spec(), _b_spec(), _w_spec(), _b_spec(),
                  _w_spec(), _w_spec()],
        out_specs=[_row_spec(_NBLK)] * 3,
        out_shape=[jax.ShapeDtypeStruct((N_NODES, D), _f32)] * 3,
    )(nodes, agg2, wn, wa, b1, w2, b2, a, b)


def _edge_enc_body(g_ref, b1_ref, w2_ref, b2_ref, out_ref):
    h = jax.nn.gelu(g_ref[...] + b1_ref[...])
    out_ref[...] = _dot(h, w2_ref[...]) + b2_ref[...]


def _edge_enc(g, b1, w2, b2):
    return pl.pallas_call(
        _edge_enc_body,
        grid=(_EGRID,),
        in_specs=[_row_spec(_EBLK), _b_spec(), _w_spec(), _b_spec()],
        out_specs=_row_spec(_EBLK),
        out_shape=jax.ShapeDtypeStruct((N_EDGES, D), _f32),
    )(g, b1, w2, b2)


def _edge_layer_body(e_ref, g_ref, c_ref, b1_ref, w2_ref, b2_ref, out_ref):
    e = e_ref[...]
    h = jax.nn.gelu(g_ref[...] + _dot(e, c_ref[...]) + b1_ref[...])
    out_ref[...] = _dot(h, w2_ref[...]) + b2_ref[...] + e


def _edge_layer(edges, g, c, b1, w2, b2):
    return pl.pallas_call(
        _edge_layer_body,
        grid=(_EGRID,),
        in_specs=[_row_spec(_EBLK), _row_spec(_EBLK), _w_spec(), _b_spec(),
                  _w_spec(), _b_spec()],
        out_specs=_row_spec(_EBLK),
        out_shape=jax.ShapeDtypeStruct((N_EDGES, D), _f32),
    )(edges, g, c, b1, w2, b2)


def kernel(node_attr, graph, params):
    src = graph[0].astype(jnp.int32)
    dst = graph[1].astype(jnp.int32)
    src3 = src.reshape(_NW, _GIPT, _GCH)
    dst3 = dst.reshape(_NW, _GIPT, _GCH)
    enc = params["node_enc"]
    ee = params["edge_enc"]

    nodes, p, q = _node_enc(
        node_attr, enc["W1"], enc["b1"].reshape(1, D), enc["W2"],
        enc["b2"].reshape(1, D), ee["W1"][:D], ee["W1"][D:])
    g = _gather_add(p, q, src3, dst3)
    edges = _edge_enc(g, ee["b1"].reshape(1, D), ee["W2"],
                      ee["b2"].reshape(1, D))

    zeros = jnp.zeros((N_NODES, D), _f32)
    for lp in params["layers"]:
        np_ = lp["node_net"]
        ep_ = lp["edge_net"]
        agg2 = _scatter_add(edges, dst3, zeros)
        nodes, p, q = _node_layer(
            nodes, agg2, np_["W1"][:D], np_["W1"][D:],
            np_["b1"].reshape(1, D), np_["W2"], np_["b2"].reshape(1, D),
            ep_["W1"][:D], ep_["W1"][D:2 * D])
        g = _gather_add(p, q, src3, dst3)
        edges = _edge_layer(edges, g, ep_["W1"][2 * D:],
                            ep_["b1"].reshape(1, D), ep_["W2"],
                            ep_["b2"].reshape(1, D))
    return (nodes, edges)


# speedup vs baseline: 1.1077x; 1.0147x over previous
import jax
import jax.numpy as jnp
from jax import lax
from jax.experimental import pallas as pl
from jax.experimental.pallas import tpu as pltpu
from jax.experimental.pallas import tpu_sc as plsc

N_NODES = 10000
N_EDGES = 320000
D = 128
_f32 = jnp.float32

_NC, _NS = 2, 16
_NW = _NC * _NS
_EPW = N_EDGES // _NW

_GCH = 80
_GCHUNK = _GCH
_GNCH = _EPW // _GCHUNK
_GIPT = _GNCH

_SCH = 80
_SIPT = _EPW // _SCH


def _sc_mesh():
    return plsc.VectorSubcoreMesh(
        core_axis_name="c", subcore_axis_name="s", num_cores=_NC, num_subcores=_NS
    )


def _gather_add_body(p_hbm, q_hbm, src3_hbm, dst3_hbm, g_hbm,
                     idx_s, idx_d, pA, pB, qA, qB, semA, semB, semwA, semwB):
    c = lax.axis_index("c")
    s = lax.axis_index("s")
    wid = s * _NC + c
    base = wid * _EPW

    pltpu.sync_copy(src3_hbm.at[wid], idx_s)
    pltpu.sync_copy(dst3_hbm.at[wid], idx_d)

    def g_start(j, bufp, bufq, sem):
        pltpu.async_copy(p_hbm.at[idx_s.at[j]], bufp, sem)
        pltpu.async_copy(q_hbm.at[idx_d.at[j]], bufq, sem)

    def g_wait(bufp, bufq, sem):
        pltpu.make_async_copy(g_hbm.at[pl.ds(0, _GCHUNK)], bufp, sem).wait()
        pltpu.make_async_copy(g_hbm.at[pl.ds(0, _GCHUNK)], bufq, sem).wait()

    def add(bufp, bufq):
        def rows(r, carry):
            for u in range(2):
                for v in range(D // 16):
                    sl = pl.ds(v * 16, 16)
                    plsc.addupdate(bufp.at[2 * r + u, sl], bufq[2 * r + u, sl])
            return carry

        lax.fori_loop(0, _GCHUNK // 2, rows, 0)

    def wb_start(gc, bufp, semw):
        pltpu.async_copy(bufp, g_hbm.at[pl.ds(base + gc * _GCHUNK, _GCHUNK)],
                         semw)

    def wb_wait(bufp, semw):
        pltpu.make_async_copy(bufp, g_hbm.at[pl.ds(0, _GCHUNK)], semw).wait()

    g_start(0, pA, qA, semA)
    g_start(1, pB, qB, semB)

    def body(t, carry):
        c0 = 2 * t
        g_wait(pA, qA, semA)
        add(pA, qA)
        wb_start(c0, pA, semwA)
        g_wait(pB, qB, semB)
        add(pB, qB)
        wb_start(c0 + 1, pB, semwB)
        wb_wait(pA, semwA)
        g_start(c0 + 2, pA, qA, semA)

        @pl.when(c0 + 3 < _GNCH)
        def _():
            wb_wait(pB, semwB)
            g_start(c0 + 3, pB, qB, semB)

        return carry

    lax.fori_loop(0, _GNCH // 2, body, 0)
    g_wait(pA, qA, semA)
    add(pA, qA)
    wb_start(_GNCH - 1, pA, semwA)
    wb_wait(pB, semwB)
    wb_wait(pA, semwA)


def _gather_add(p, q, src3, dst3):
    f = pl.kernel(
        _gather_add_body,
        out_type=jax.ShapeDtypeStruct((N_EDGES, D), _f32),
        mesh=_sc_mesh(),
        scratch_types=(
            pltpu.VMEM((_GIPT, _GCH), jnp.int32),
            pltpu.VMEM((_GIPT, _GCH), jnp.int32),
            pltpu.VMEM((_GCHUNK, D), _f32),
            pltpu.VMEM((_GCHUNK, D), _f32),
            pltpu.VMEM((_GCHUNK, D), _f32),
            pltpu.VMEM((_GCHUNK, D), _f32),
            pltpu.SemaphoreType.DMA,
            pltpu.SemaphoreType.DMA,
            pltpu.SemaphoreType.DMA,
            pltpu.SemaphoreType.DMA,
        ),
    )
    return f(p, q, src3, dst3)


def _scatter_body(edges_hbm, dst3_hbm, zeros_hbm, out_hbm,
                  idx_d, bufA, bufB, acc, semA, semB):
    c = lax.axis_index("c")
    s = lax.axis_index("s")
    wid = c * _NS + s
    base = wid * _EPW

    pltpu.sync_copy(dst3_hbm.at[wid], idx_d)

    @pl.when(s == 0)
    def _():
        pltpu.sync_copy(zeros_hbm, acc)

    plsc.subcore_barrier()

    def r_start(ch, buf, sem):
        pltpu.async_copy(edges_hbm.at[pl.ds(base + ch * _SCH, _SCH)], buf, sem)

    def r_wait(buf, sem):
        pltpu.make_async_copy(edges_hbm.at[pl.ds(0, _SCH)], buf, sem).wait()

    def scat(ch, buf):
        pltpu.sync_copy(buf, acc.at[idx_d.at[ch]], add=True)

    r_start(0, bufA, semA)
    r_start(1, bufB, semB)

    def body(t, carry):
        c0 = 2 * t
        r_wait(bufA, semA)
        scat(c0, bufA)
        r_start(c0 + 2, bufA, semA)
        r_wait(bufB, semB)
        scat(c0 + 1, bufB)

        @pl.when(c0 + 3 < _SIPT)
        def _():
            r_start(c0 + 3, bufB, semB)

        return carry

    lax.fori_loop(0, _SIPT // 2, body, 0)
    r_wait(bufA, semA)
    scat(_SIPT - 1, bufA)

    plsc.subcore_barrier()

    @pl.when(s == 0)
    def _():
        pltpu.sync_copy(acc, out_hbm.at[c])


def _scatter_add(edges, dst3, zeros):
    f = pl.kernel(
        _scatter_body,
        out_type=jax.ShapeDtypeStruct((_NC, N_NODES, D), _f32),
        mesh=_sc_mesh(),
        scratch_types=(
            pltpu.VMEM((_SIPT, _SCH), jnp.int32),
            pltpu.VMEM((_SCH, D), _f32),
            pltpu.VMEM((_SCH, D), _f32),
            pltpu.VMEM_SHARED((N_NODES, D), _f32),
            pltpu.SemaphoreType.DMA,
            pltpu.SemaphoreType.DMA,
        ),
    )
    return f(edges, dst3, zeros)


_NBLK = 1000
_NGRID = N_NODES // _NBLK
_EBLK = 16000
_EGRID = N_EDGES // _EBLK


def _row_spec(blk):
    return pl.BlockSpec((blk, D), lambda i: (i, 0))


def _w_spec():
    return pl.BlockSpec((D, D), lambda i: (0, 0))


def _b_spec():
    return pl.BlockSpec((1, D), lambda i: (0, 0))


def _dot(a, b):
    return jnp.dot(a, b, preferred_element_type=_f32)


def _enc_body(x_ref, w1_ref, b1_ref, w2_ref, b2_ref, a_ref, b_ref,
              nodes_ref, p_ref, q_ref):
    h = jax.nn.gelu(_dot(x_ref[...], w1_ref[...]) + b1_ref[...])
    n = _dot(h, w2_ref[...]) + b2_ref[...]
    nodes_ref[...] = n
    p_ref[...] = _dot(n, a_ref[...])
    q_ref[...] = _dot(n, b_ref[...])


def _node_enc(x, w1, b1, w2, b2, a, b):
    return pl.pallas_call(
        _enc_body,
        grid=(_NGRID,),
        in_specs=[_row_spec(_NBLK), _w_spec(), _b_spec(), _w_spec(), _b_spec(),
                  _w_spec(), _w_spec()],
        out_specs=[_row_spec(_NBLK)] * 3,
        out_shape=[jax.ShapeDtypeStruct((N_NODES, D), _f32)] * 3,
    )(x, w1, b1, w2, b2, a, b)


def _node_layer_body(n_ref, agg_ref, wn_ref, wa_ref, b1_ref, w2_ref, b2_ref,
                     a_ref, b_ref, out_ref, p_ref, q_ref):
    n = n_ref[...]
    agg = agg_ref[0] + agg_ref[1]
    h = jax.nn.gelu(_dot(n, wn_ref[...]) + _dot(agg, wa_ref[...]) + b1_ref[...])
    nn = _dot(h, w2_ref[...]) + b2_ref[...] + n
    out_ref[...] = nn
    p_ref[...] = _dot(nn, a_ref[...])
    q_ref[...] = _dot(nn, b_ref[...])


def _node_layer(nodes, agg2, wn, wa, b1, w2, b2, a, b):
    return pl.pallas_call(
        _node_layer_body,
        grid=(_NGRID,),
        in_specs=[_row_spec(_NBLK),
                  pl.BlockSpec((_NC, _NBLK, D), lambda i: (0, i, 0)),
                  _w_spec(), _w_spec(), _b_spec(), _w_spec(), _b_spec(),
                  _w_spec(), _w_spec()],
        out_specs=[_row_spec(_NBLK)] * 3,
        out_shape=[jax.ShapeDtypeStruct((N_NODES, D), _f32)] * 3,
    )(nodes, agg2, wn, wa, b1, w2, b2, a, b)


def _edge_enc_body(g_ref, b1_ref, w2_ref, b2_ref, out_ref):
    h = jax.nn.gelu(g_ref[...] + b1_ref[...])
    out_ref[...] = _dot(h, w2_ref[...]) + b2_ref[...]


def _edge_enc(g, b1, w2, b2):
    return pl.pallas_call(
        _edge_enc_body,
        grid=(_EGRID,),
        in_specs=[_row_spec(_EBLK), _b_spec(), _w_spec(), _b_spec()],
        out_specs=_row_spec(_EBLK),
        out_shape=jax.ShapeDtypeStruct((N_EDGES, D), _f32),
    )(g, b1, w2, b2)


def _edge_layer_body(e_ref, g_ref, c_ref, b1_ref, w2_ref, b2_ref, out_ref):
    e = e_ref[...]
    h = jax.nn.gelu(g_ref[...] + _dot(e, c_ref[...]) + b1_ref[...])
    out_ref[...] = _dot(h, w2_ref[...]) + b2_ref[...] + e


def _edge_layer(edges, g, c, b1, w2, b2):
    return pl.pallas_call(
        _edge_layer_body,
        grid=(_EGRID,),
        in_specs=[_row_spec(_EBLK), _row_spec(_EBLK), _w_spec(), _b_spec(),
                  _w_spec(), _b_spec()],
        out_specs=_row_spec(_EBLK),
        out_shape=jax.ShapeDtypeStruct((N_EDGES, D), _f32),
    )(edges, g, c, b1, w2, b2)


def kernel(node_attr, graph, params):
    src = graph[0].astype(jnp.int32)
    dst = graph[1].astype(jnp.int32)
    src3 = src.reshape(_NW, _GIPT, _GCH)
    dst3 = dst.reshape(_NW, _GIPT, _GCH)
    enc = params["node_enc"]
    ee = params["edge_enc"]

    nodes, p, q = _node_enc(
        node_attr, enc["W1"], enc["b1"].reshape(1, D), enc["W2"],
        enc["b2"].reshape(1, D), ee["W1"][:D], ee["W1"][D:])
    g = _gather_add(p, q, src3, dst3)
    edges = _edge_enc(g, ee["b1"].reshape(1, D), ee["W2"],
                      ee["b2"].reshape(1, D))

    zeros = jnp.zeros((N_NODES, D), _f32)
    for lp in params["layers"]:
        np_ = lp["node_net"]
        ep_ = lp["edge_net"]
        agg2 = _scatter_add(edges, dst3, zeros)
        nodes, p, q = _node_layer(
            nodes, agg2, np_["W1"][:D], np_["W1"][D:],
            np_["b1"].reshape(1, D), np_["W2"], np_["b2"].reshape(1, D),
            ep_["W1"][:D], ep_["W1"][D:2 * D])
        g = _gather_add(p, q, src3, dst3)
        edges = _edge_layer(edges, g, ep_["W1"][2 * D:],
                            ep_["b1"].reshape(1, D), ep_["W2"],
                            ep_["b2"].reshape(1, D))
    return (nodes, edges)
